# baseline (speedup 1.0000x reference)
import functools

import jax
import jax.numpy as jnp
from jax import lax
from jax.experimental import pallas as pl
from jax.experimental.pallas import tpu as pltpu
from jax.experimental.pallas import tpu_sc as plsc

B, N, P, D = 8, 128, 512, 64
K = 52
KPAD = 64
NBLK = 16
NCHUNKS = N // NBLK
NC_SC, NS_SC = 2, 16
NTILES = NC_SC * NS_SC
PAIRS_PER_TILE = (B * N) // NTILES
TILES_PER_B = NTILES // B
LANES = 16
NEG = -3.402823e38


def _dense_body(x_ref, sumsq_ref, meanp_ref):
    nc = pl.program_id(1)
    x = x_ref[0]
    sumsq_ref[...] = jnp.sum(x * x, axis=-1)[None]
    m_chunk = jnp.sum(x, axis=0)[None]

    @pl.when(nc == 0)
    def _():
        meanp_ref[...] = m_chunk

    @pl.when(nc != 0)
    def _():
        meanp_ref[...] += m_chunk

    @pl.when(nc == NCHUNKS - 1)
    def _():
        meanp_ref[...] *= jnp.float32(1.0 / N)


def _dense_pass(patches):
    return pl.pallas_call(
        _dense_body,
        grid=(B, NCHUNKS),
        in_specs=[
            pl.BlockSpec((1, NBLK, P, D), lambda b, nc: (b, nc, 0, 0)),
        ],
        out_specs=[
            pl.BlockSpec((1, NBLK, P), lambda b, nc: (b, nc, 0)),
            pl.BlockSpec((1, P, D), lambda b, nc: (b, 0, 0)),
        ],
        out_shape=[
            jax.ShapeDtypeStruct((B, N, P), jnp.float32),
            jax.ShapeDtypeStruct((B, P, D), jnp.float32),
        ],
    )(patches)


def _sc_body(scores_hbm, meanp_hbm, out_hbm, vals_v, idx_v, rows_v, sem):
    wid = lax.axis_index("c") * NS_SC + lax.axis_index("s")
    b = wid // TILES_PER_B
    base = b * P
    pltpu.sync_copy(scores_hbm.at[b], vals_v)

    iota = lax.iota(jnp.int32, LANES)
    lane0 = iota == 0
    for jj in range(KPAD // LANES):
        idx_v[pl.ds(jj * LANES, LANES)] = jnp.full((LANES,), base, jnp.int32)

    def topk_body(k, carry):
        def mx(j, vm):
            return jnp.maximum(vm, vals_v[pl.ds(j * LANES, LANES)])
        vm = lax.fori_loop(0, P // LANES, mx,
                           jnp.full((LANES,), NEG, jnp.float32))
        m = jnp.max(vm)

        def fidx(j, vi):
            v = vals_v[pl.ds(j * LANES, LANES)]
            cand = jnp.where(v == m, iota + j * LANES, jnp.int32(1 << 30))
            return jnp.minimum(vi, cand)
        vi = lax.fori_loop(0, P // LANES, fidx,
                           jnp.full((LANES,), 1 << 30, jnp.int32))
        pidx = jnp.min(vi)

        pv = jnp.full((LANES,), pidx, jnp.int32)
        plsc.store_scatter(vals_v, [pv],
                           jnp.full((LANES,), NEG, jnp.float32), mask=lane0)
        plsc.store_scatter(idx_v, [jnp.full((LANES,), k, jnp.int32)],
                           pv + base, mask=lane0)
        return carry

    lax.fori_loop(0, K, topk_body, jnp.int32(0))

    pltpu.async_copy(meanp_hbm.at[idx_v], rows_v, sem).wait()

    def wr(i, carry):
        pltpu.sync_copy(rows_v.at[pl.ds(0, K)],
                        out_hbm.at[wid * PAIRS_PER_TILE + i])
        return carry

    lax.fori_loop(0, PAIRS_PER_TILE, wr, jnp.int32(0))


@functools.cache
def _sc_select():
    return functools.partial(
        pl.kernel,
        out_type=jax.ShapeDtypeStruct((B * N, K, D), jnp.float32),
        mesh=plsc.VectorSubcoreMesh(core_axis_name="c", subcore_axis_name="s",
                                    num_cores=NC_SC, num_subcores=NS_SC),
        compiler_params=pltpu.CompilerParams(needs_layout_passes=False,
                                             use_tc_tiling_on_sc=False),
        scratch_types=[
            pltpu.VMEM((P,), jnp.float32),
            pltpu.VMEM((KPAD,), jnp.int32),
            pltpu.VMEM((KPAD, D), jnp.float32),
            pltpu.SemaphoreType.DMA,
        ],
    )(_sc_body)


def kernel(patches, adp):
    sumsq, meanp = _dense_pass(patches)
    importance = adp.mean(axis=0)
    norms = jnp.sqrt(sumsq)
    scores = jnp.einsum('bnp,n->bp', norms, importance)
    return _sc_select()(scores, meanp.reshape(B * P, D))


# speedup vs baseline: 2.5052x; 2.5052x over previous
import functools

import jax
import jax.numpy as jnp
from jax import lax
from jax.experimental import pallas as pl
from jax.experimental.pallas import tpu as pltpu
from jax.experimental.pallas import tpu_sc as plsc

B, N, P, D = 8, 128, 512, 64
K = 52
KPAD = 64
NBLK = 16
NCHUNKS = N // NBLK
NC_SC, NS_SC = 2, 16
NTILES = NC_SC * NS_SC
PAIRS_PER_TILE = (B * N) // NTILES
TILES_PER_B = NTILES // B
LANES = 16
NEG = -3.402823e38


def _dense_body(x_ref, sumsq_ref, meanp_ref):
    nc = pl.program_id(1)
    x = x_ref[0]
    sumsq_ref[...] = jnp.sum(x * x, axis=1)[None]
    m_chunk = jnp.sum(x, axis=0)[None]

    @pl.when(nc == 0)
    def _():
        meanp_ref[...] = m_chunk

    @pl.when(nc != 0)
    def _():
        meanp_ref[...] += m_chunk

    @pl.when(nc == NCHUNKS - 1)
    def _():
        meanp_ref[...] *= jnp.float32(1.0 / N)


def _dense_pass(patches_t):
    return pl.pallas_call(
        _dense_body,
        grid=(B, NCHUNKS),
        in_specs=[
            pl.BlockSpec((1, NBLK, D, P), lambda b, nc: (b, nc, 0, 0)),
        ],
        out_specs=[
            pl.BlockSpec((1, NBLK, P), lambda b, nc: (b, nc, 0)),
            pl.BlockSpec((1, D, P), lambda b, nc: (b, 0, 0)),
        ],
        out_shape=[
            jax.ShapeDtypeStruct((B, N, P), jnp.float32),
            jax.ShapeDtypeStruct((B, D, P), jnp.float32),
        ],
    )(patches_t)


def _sc_body(scores_hbm, meanp_hbm, out_hbm, vals_v, idx_v, rows_v, sem):
    wid = lax.axis_index("c") * NS_SC + lax.axis_index("s")
    b = wid // TILES_PER_B
    base = b * P
    pltpu.sync_copy(scores_hbm.at[b], vals_v)

    iota = lax.iota(jnp.int32, LANES)
    lane0 = iota == 0
    for jj in range(KPAD // LANES):
        idx_v[pl.ds(jj * LANES, LANES)] = jnp.full((LANES,), base, jnp.int32)

    def topk_body(k, carry):
        def mx(j, vm):
            return jnp.maximum(vm, vals_v[pl.ds(j * LANES, LANES)])
        vm = lax.fori_loop(0, P // LANES, mx,
                           jnp.full((LANES,), NEG, jnp.float32))
        m = jnp.max(vm)

        def fidx(j, vi):
            v = vals_v[pl.ds(j * LANES, LANES)]
            cand = jnp.where(v == m, iota + j * LANES, jnp.int32(1 << 30))
            return jnp.minimum(vi, cand)
        vi = lax.fori_loop(0, P // LANES, fidx,
                           jnp.full((LANES,), 1 << 30, jnp.int32))
        pidx = jnp.min(vi)

        pv = jnp.full((LANES,), pidx, jnp.int32)
        plsc.store_scatter(vals_v, [pv],
                           jnp.full((LANES,), NEG, jnp.float32), mask=lane0)
        plsc.store_scatter(idx_v, [jnp.full((LANES,), k, jnp.int32)],
                           pv + base, mask=lane0)
        return carry

    lax.fori_loop(0, K, topk_body, jnp.int32(0))

    pltpu.async_copy(meanp_hbm.at[idx_v], rows_v, sem).wait()

    def wr(i, carry):
        pltpu.sync_copy(rows_v.at[pl.ds(0, K)],
                        out_hbm.at[wid * PAIRS_PER_TILE + i])
        return carry

    lax.fori_loop(0, PAIRS_PER_TILE, wr, jnp.int32(0))


@functools.cache
def _sc_select():
    return functools.partial(
        pl.kernel,
        out_type=jax.ShapeDtypeStruct((B * N, K, D), jnp.float32),
        mesh=plsc.VectorSubcoreMesh(core_axis_name="c", subcore_axis_name="s",
                                    num_cores=NC_SC, num_subcores=NS_SC),
        compiler_params=pltpu.CompilerParams(needs_layout_passes=False,
                                             use_tc_tiling_on_sc=False),
        scratch_types=[
            pltpu.VMEM((P,), jnp.float32),
            pltpu.VMEM((KPAD,), jnp.int32),
            pltpu.VMEM((KPAD, D), jnp.float32),
            pltpu.SemaphoreType.DMA,
        ],
    )(_sc_body)


def kernel(patches, adp):
    sumsq, meanp_t = _dense_pass(jnp.transpose(patches, (0, 1, 3, 2)))
    importance = adp.mean(axis=0)
    norms = jnp.sqrt(sumsq)
    scores = jnp.einsum('bnp,n->bp', norms, importance)
    meanp = jnp.transpose(meanp_t, (0, 2, 1)).reshape(B * P, D)
    return _sc_select()(scores, meanp)


# speedup vs baseline: 2.9001x; 1.1576x over previous
import functools

import jax
import jax.numpy as jnp
from jax import lax
from jax.experimental import pallas as pl
from jax.experimental.pallas import tpu as pltpu
from jax.experimental.pallas import tpu_sc as plsc

B, N, P, D = 8, 128, 512, 64
K = 52
KPAD = 64
NBLK = 16
NCHUNKS = N // NBLK
NC_SC, NS_SC = 2, 16
NTILES = NC_SC * NS_SC
PAIRS_PER_TILE = (B * N) // NTILES
TILES_PER_B = NTILES // B
LANES = 16
NEG = -3.402823e38


def _dense_body(x_ref, sumsq_ref, meanp_ref):
    nc = pl.program_id(1)
    x = x_ref[0]
    sumsq_ref[...] = jnp.sum(x * x, axis=1)[None]
    m_chunk = jnp.sum(x, axis=0)[None]

    @pl.when(nc == 0)
    def _():
        meanp_ref[...] = m_chunk

    @pl.when(nc != 0)
    def _():
        meanp_ref[...] += m_chunk

    @pl.when(nc == NCHUNKS - 1)
    def _():
        meanp_ref[...] *= jnp.float32(1.0 / N)


def _dense_pass(patches_t):
    return pl.pallas_call(
        _dense_body,
        grid=(B, NCHUNKS),
        in_specs=[
            pl.BlockSpec((1, NBLK, D, P), lambda b, nc: (b, nc, 0, 0)),
        ],
        out_specs=[
            pl.BlockSpec((1, NBLK, P), lambda b, nc: (b, nc, 0)),
            pl.BlockSpec((1, D, P), lambda b, nc: (b, 0, 0)),
        ],
        out_shape=[
            jax.ShapeDtypeStruct((B, N, P), jnp.float32),
            jax.ShapeDtypeStruct((B, D, P), jnp.float32),
        ],
    )(patches_t)


def _sc_body(scores_hbm, meanp_hbm, out_hbm, vals_v, idx_v, rows_v, sem):
    wid = lax.axis_index("c") * NS_SC + lax.axis_index("s")
    b = wid // TILES_PER_B

    @pl.when(wid % TILES_PER_B == 0)
    def _():
        base = b * P
        pltpu.sync_copy(scores_hbm.at[b], vals_v)

        iota = lax.iota(jnp.int32, LANES)
        lane0 = iota == 0
        for jj in range(KPAD // LANES):
            idx_v[pl.ds(jj * LANES, LANES)] = jnp.full((LANES,), base,
                                                       jnp.int32)

        def topk_body(k, carry):
            def mx(j, vm):
                return jnp.maximum(vm, vals_v[pl.ds(j * LANES, LANES)])
            vm = lax.fori_loop(0, P // LANES, mx,
                               jnp.full((LANES,), NEG, jnp.float32))
            m = jnp.max(vm)

            def fidx(j, vi):
                v = vals_v[pl.ds(j * LANES, LANES)]
                cand = jnp.where(v == m, iota + j * LANES, jnp.int32(1 << 30))
                return jnp.minimum(vi, cand)
            vi = lax.fori_loop(0, P // LANES, fidx,
                               jnp.full((LANES,), 1 << 30, jnp.int32))
            pidx = jnp.min(vi)

            pv = jnp.full((LANES,), pidx, jnp.int32)
            plsc.store_scatter(vals_v, [pv],
                               jnp.full((LANES,), NEG, jnp.float32),
                               mask=lane0)
            plsc.store_scatter(idx_v, [jnp.full((LANES,), k, jnp.int32)],
                               pv + base, mask=lane0)
            return carry

        lax.fori_loop(0, K, topk_body, jnp.int32(0))

        pltpu.async_copy(meanp_hbm.at[idx_v], rows_v, sem).wait()
        pltpu.sync_copy(rows_v.at[pl.ds(0, K)], out_hbm.at[b])


@functools.cache
def _sc_select():
    return functools.partial(
        pl.kernel,
        out_type=jax.ShapeDtypeStruct((B, K, D), jnp.float32),
        mesh=plsc.VectorSubcoreMesh(core_axis_name="c", subcore_axis_name="s",
                                    num_cores=NC_SC, num_subcores=NS_SC),
        compiler_params=pltpu.CompilerParams(needs_layout_passes=False,
                                             use_tc_tiling_on_sc=False),
        scratch_types=[
            pltpu.VMEM((P,), jnp.float32),
            pltpu.VMEM((KPAD,), jnp.int32),
            pltpu.VMEM((KPAD, D), jnp.float32),
            pltpu.SemaphoreType.DMA,
        ],
    )(_sc_body)


def kernel(patches, adp):
    sumsq, meanp_t = _dense_pass(jnp.transpose(patches, (0, 1, 3, 2)))
    importance = adp.mean(axis=0)
    norms = jnp.sqrt(sumsq)
    scores = jnp.einsum('bnp,n->bp', norms, importance)
    meanp = jnp.transpose(meanp_t, (0, 2, 1)).reshape(B * P, D)
    anchors = _sc_select()(scores, meanp)
    return jnp.broadcast_to(anchors[:, None], (B, N, K, D)).reshape(B * N, K, D)
